# baseline (speedup 1.0000x reference)
import functools

import jax
import jax.numpy as jnp
from jax import lax
from jax.experimental import pallas as pl
from jax.experimental.pallas import tpu as pltpu
from jax.experimental.pallas import tpu_sc as plsc

_N = 10000
_D = 128
_NC = 2
_NS = 16
_NW = _NC * _NS
_E_TOT = 320000
_CHUNK = 128
_NCHUNK = 80
_EPW = _NCHUNK * _CHUNK
_E_PAD = _EPW * _NW
_N_PAD = 10112
_RPS = _N_PAD // _NS
_PAGE = 20
_NPAGE = _NCHUNK // _PAGE
_N_PAD_DEG = 10240
_RPS_DEG = _N_PAD_DEG // _NS

_BLK = 1000
_GRID = _N // _BLK

_mesh = plsc.VectorSubcoreMesh(
    core_axis_name="c", subcore_axis_name="s", num_cores=_NC, num_subcores=_NS)


@functools.partial(
    pl.kernel,
    out_type=jax.ShapeDtypeStruct((_NC, _N_PAD_DEG), jnp.float32),
    mesh=_mesh,
    scratch_types=[
        pltpu.VMEM((_NCHUNK, _CHUNK), jnp.int32),
        pltpu.VMEM((_CHUNK,), jnp.float32),
        pltpu.VMEM_SHARED((_N_PAD_DEG,), jnp.float32),
    ],
)
def _sc_degree(dst_hbm, zeros1_hbm, out_hbm, didx, ones, dacc):
    c = lax.axis_index("c")
    s = lax.axis_index("s")
    wid = s * _NC + c
    pltpu.sync_copy(dst_hbm.at[wid], didx)
    for i in range(_CHUNK // 16):
        ones[pl.ds(i * 16, 16)] = jnp.ones((16,), jnp.float32)
    pltpu.sync_copy(zeros1_hbm.at[pl.ds(s * _RPS_DEG, _RPS_DEG)],
                    dacc.at[pl.ds(s * _RPS_DEG, _RPS_DEG)])
    plsc.subcore_barrier()

    def body(j, carry):
        pltpu.sync_copy(ones, dacc.at[didx.at[j]], add=True)
        return carry

    lax.fori_loop(0, _NCHUNK, body, 0)
    plsc.subcore_barrier()
    pltpu.sync_copy(dacc.at[pl.ds(s * _RPS_DEG, _RPS_DEG)],
                    out_hbm.at[c, pl.ds(s * _RPS_DEG, _RPS_DEG)])


@functools.partial(
    pl.kernel,
    out_type=jax.ShapeDtypeStruct((_NC, _N_PAD, _D), jnp.float32),
    mesh=_mesh,
    scratch_types=[
        pltpu.VMEM((_PAGE, _CHUNK), jnp.int32),
        pltpu.VMEM((_PAGE, _CHUNK), jnp.int32),
        pltpu.VMEM((_CHUNK, _D), jnp.float32),
        pltpu.VMEM((_CHUNK, _D), jnp.float32),
        pltpu.SemaphoreType.DMA,
        pltpu.SemaphoreType.DMA,
        pltpu.VMEM_SHARED((_N_PAD, _D), jnp.float32),
    ],
)
def _sc_message(src_hbm0, src_hbm1, src_hbm2, src_hbm3,
                dst_hbm0, dst_hbm1, dst_hbm2, dst_hbm3,
                xs_hbm, zeros_hbm, out_hbm,
                sidx, didx, g0, g1, sem0, sem1, acc):
    src_pages = (src_hbm0, src_hbm1, src_hbm2, src_hbm3)
    dst_pages = (dst_hbm0, dst_hbm1, dst_hbm2, dst_hbm3)
    c = lax.axis_index("c")
    s = lax.axis_index("s")
    wid = s * _NC + c
    pltpu.sync_copy(zeros_hbm.at[pl.ds(s * _RPS, _RPS)],
                    acc.at[pl.ds(s * _RPS, _RPS)])
    plsc.subcore_barrier()

    for p in range(_NPAGE):
        pltpu.sync_copy(src_pages[p].at[wid], sidx)
        pltpu.sync_copy(dst_pages[p].at[wid], didx)

        def body(t, carry):
            pltpu.async_copy(xs_hbm.at[sidx.at[t]], g0, sem0).wait()
            pltpu.sync_copy(g0, acc.at[didx.at[t]], add=True)
            return carry

        lax.fori_loop(0, _PAGE, body, 0)
    plsc.subcore_barrier()
    pltpu.sync_copy(acc.at[pl.ds(s * _RPS, _RPS)],
                    out_hbm.at[c, pl.ds(s * _RPS, _RPS)])


def _type_of(i):
    return jnp.where(i < 4, 0, jnp.where(i < 7, 1, 2))


def _tc1_body(x_ref, w_ref, b_ref, d0_ref, d1_ref, x1_ref, xs_ref, rdeg_ref):
    deg = d0_ref[0, 0] + d1_ref[0, 0] + 1.0
    rdeg = lax.rsqrt(deg)
    rdeg_ref[0, 0] = rdeg
    x1 = jnp.dot(x_ref[...], w_ref[0], preferred_element_type=jnp.float32)
    x1 = x1 + b_ref[0]
    x1_ref[...] = x1
    xs_ref[...] = x1 * rdeg[:, None]


def _tc2_body(x1_ref, p0_ref, p1_ref, rdeg_ref, w_ref, b_ref, x2_ref, xs_ref):
    rdeg = rdeg_ref[0, 0][:, None]
    h = jnp.maximum(x1_ref[...] + rdeg * (p0_ref[...] + p1_ref[...]), 0.0)
    x2 = jnp.dot(h, w_ref[0], preferred_element_type=jnp.float32) + b_ref[0]
    x2_ref[...] = x2
    xs_ref[...] = x2 * rdeg


def _tc3_body(x2_ref, p0_ref, p1_ref, rdeg_ref, out_ref):
    rdeg = rdeg_ref[0, 0][:, None]
    out_ref[...] = x2_ref[...] + rdeg * (p0_ref[...] + p1_ref[...])


_row_spec = pl.BlockSpec((_BLK, _D), lambda i: (i, 0))
_w_spec = pl.BlockSpec((1, _D, _D), lambda i: (_type_of(i), 0, 0))
_b_spec = pl.BlockSpec((1, 1, _D), lambda i: (_type_of(i), 0, 0))
_vec_spec = pl.BlockSpec((1, 1, _BLK), lambda i: (i, 0, 0))

_tc1 = pl.pallas_call(
    _tc1_body,
    grid=(_GRID,),
    in_specs=[_row_spec, _w_spec, _b_spec, _vec_spec, _vec_spec],
    out_specs=[_row_spec, _row_spec, _vec_spec],
    out_shape=[
        jax.ShapeDtypeStruct((_N, _D), jnp.float32),
        jax.ShapeDtypeStruct((_N, _D), jnp.float32),
        jax.ShapeDtypeStruct((_GRID, 1, _BLK), jnp.float32),
    ],
)

_tc2 = pl.pallas_call(
    _tc2_body,
    grid=(_GRID,),
    in_specs=[_row_spec, _row_spec, _row_spec, _vec_spec, _w_spec, _b_spec],
    out_specs=[_row_spec, _row_spec],
    out_shape=[
        jax.ShapeDtypeStruct((_N, _D), jnp.float32),
        jax.ShapeDtypeStruct((_N, _D), jnp.float32),
    ],
)

_tc3 = pl.pallas_call(
    _tc3_body,
    grid=(_GRID,),
    in_specs=[_row_spec, _row_spec, _row_spec, _vec_spec],
    out_specs=_row_spec,
    out_shape=jax.ShapeDtypeStruct((_N, _D), jnp.float32),
)


def kernel(x0, x1, x2, edge_index0, edge_index1, edge_index2,
           W0_0, b0_0, W0_1, b0_1, W0_2, b0_2,
           W1_0, b1_0, W1_1, b1_1, W1_2, b1_2):
    x_cat = jnp.concatenate([x0, x1, x2], axis=0)
    W0 = jnp.stack([W0_0, W0_1, W0_2])
    b0 = jnp.stack([b0_0, b0_1, b0_2])[:, None, :]
    W1 = jnp.stack([W1_0, W1_1, W1_2])
    b1 = jnp.stack([b1_0, b1_1, b1_2])[:, None, :]

    src = jnp.concatenate([edge_index0[0], edge_index1[0], edge_index2[0]])
    dst = jnp.concatenate([edge_index0[1], edge_index1[1], edge_index2[1]])
    npad = _E_PAD - _E_TOT
    src_p = jnp.concatenate([src, jnp.zeros((npad,), jnp.int32)])
    dst_p = jnp.concatenate([dst, jnp.full((npad,), _N_PAD - 1, jnp.int32)])
    src_w = src_p.reshape(_NW, _NCHUNK, _CHUNK)
    dst_w = dst_p.reshape(_NW, _NCHUNK, _CHUNK)
    src_pages = [src_p.reshape(_NW, _NPAGE, _PAGE, _CHUNK)[:, p]
                 for p in range(_NPAGE)]
    dst_pages = [dst_p.reshape(_NW, _NPAGE, _PAGE, _CHUNK)[:, p]
                 for p in range(_NPAGE)]

    zeros1 = jnp.zeros((_N_PAD_DEG,), jnp.float32)
    zeros2 = jnp.zeros((_N_PAD, _D), jnp.float32)

    degp = _sc_degree(dst_w, zeros1)
    d0 = degp[0, :_N].reshape(_GRID, 1, _BLK)
    d1 = degp[1, :_N].reshape(_GRID, 1, _BLK)

    x1_full, xs1, rdeg_r = _tc1(x_cat, W0, b0, d0, d1)

    p = _sc_message(*src_pages, *dst_pages, xs1, zeros2)
    x2_full, xs2 = _tc2(x1_full, p[0, :_N], p[1, :_N], rdeg_r, W1, b1)

    q = _sc_message(*src_pages, *dst_pages, xs2, zeros2)
    return _tc3(x2_full, q[0, :_N], q[1, :_N], rdeg_r)


# speedup vs baseline: 1.1858x; 1.1858x over previous
import functools

import jax
import jax.numpy as jnp
from jax import lax
from jax.experimental import pallas as pl
from jax.experimental.pallas import tpu as pltpu
from jax.experimental.pallas import tpu_sc as plsc

_N = 10000
_D = 128
_NC = 2
_NS = 16
_NW = _NC * _NS
_E_TOT = 320000
_CHUNK = 128
_NCHUNK = 80
_EPW = _NCHUNK * _CHUNK
_E_PAD = _EPW * _NW
_N_PAD = 10112
_RPS = _N_PAD // _NS
_PAGE = 20
_NPAGE = _NCHUNK // _PAGE
_N_PAD_DEG = 10240
_RPS_DEG = _N_PAD_DEG // _NS

_BLK = 1000
_GRID = _N // _BLK

_mesh = plsc.VectorSubcoreMesh(
    core_axis_name="c", subcore_axis_name="s", num_cores=_NC, num_subcores=_NS)


@functools.partial(
    pl.kernel,
    out_type=jax.ShapeDtypeStruct((_NC, _N_PAD_DEG), jnp.float32),
    mesh=_mesh,
    scratch_types=[
        pltpu.VMEM((_NCHUNK, _CHUNK), jnp.int32),
        pltpu.VMEM((_CHUNK,), jnp.float32),
        pltpu.VMEM_SHARED((_N_PAD_DEG,), jnp.float32),
    ],
)
def _sc_degree(dst_hbm, zeros1_hbm, out_hbm, didx, ones, dacc):
    c = lax.axis_index("c")
    s = lax.axis_index("s")
    wid = s * _NC + c
    pltpu.sync_copy(dst_hbm.at[wid], didx)
    for i in range(_CHUNK // 16):
        ones[pl.ds(i * 16, 16)] = jnp.ones((16,), jnp.float32)
    pltpu.sync_copy(zeros1_hbm.at[pl.ds(s * _RPS_DEG, _RPS_DEG)],
                    dacc.at[pl.ds(s * _RPS_DEG, _RPS_DEG)])
    plsc.subcore_barrier()

    def body(j, carry):
        pltpu.sync_copy(ones, dacc.at[didx.at[j]], add=True)
        return carry

    lax.fori_loop(0, _NCHUNK, body, 0)
    plsc.subcore_barrier()
    pltpu.sync_copy(dacc.at[pl.ds(s * _RPS_DEG, _RPS_DEG)],
                    out_hbm.at[c, pl.ds(s * _RPS_DEG, _RPS_DEG)])


@functools.partial(
    pl.kernel,
    out_type=jax.ShapeDtypeStruct((_NC, _N_PAD, _D), jnp.float32),
    mesh=_mesh,
    scratch_types=[
        pltpu.VMEM((_NCHUNK, _CHUNK), jnp.int32),
        pltpu.VMEM((_CHUNK,), jnp.int32),
        pltpu.VMEM((_CHUNK,), jnp.int32),
        pltpu.VMEM((_CHUNK,), jnp.int32),
        pltpu.VMEM((_CHUNK,), jnp.int32),
        pltpu.VMEM((_CHUNK, _D), jnp.float32),
        pltpu.VMEM((_CHUNK, _D), jnp.float32),
        pltpu.SemaphoreType.DMA,
        pltpu.SemaphoreType.DMA,
        pltpu.VMEM_SHARED((_N_PAD, _D), jnp.float32),
    ],
)
def _sc_message(packed_hbm, xs_hbm, zeros_hbm, out_hbm,
                pidx, sA, dA, sB, dB, g0, g1, sem0, sem1, acc):
    c = lax.axis_index("c")
    s = lax.axis_index("s")
    wid = s * _NC + c
    pltpu.sync_copy(packed_hbm.at[wid], pidx)
    pltpu.sync_copy(zeros_hbm.at[pl.ds(s * _RPS, _RPS)],
                    acc.at[pl.ds(s * _RPS, _RPS)])

    def unpack(j, srow, drow):
        for i in range(_CHUNK // 16):
            v = pidx[j, pl.ds(i * 16, 16)]
            srow[pl.ds(i * 16, 16)] = v & 0xFFFF
            drow[pl.ds(i * 16, 16)] = lax.shift_right_logical(v, 16)

    plsc.subcore_barrier()

    unpack(0, sA, dA)
    pltpu.async_copy(xs_hbm.at[sA], g0, sem0)

    def body(t, carry):
        j0 = 2 * t
        unpack(j0 + 1, sB, dB)
        pltpu.make_async_copy(xs_hbm.at[sA], g0, sem0).wait()
        pltpu.async_copy(xs_hbm.at[sB], g1, sem1)
        pltpu.sync_copy(g0, acc.at[dA], add=True)

        @pl.when(t < _NCHUNK // 2 - 1)
        def _():
            unpack(j0 + 2, sA, dA)
            pltpu.async_copy(xs_hbm.at[sA], g0, sem0)

        pltpu.make_async_copy(xs_hbm.at[sB], g1, sem1).wait()
        pltpu.sync_copy(g1, acc.at[dB], add=True)
        return carry

    lax.fori_loop(0, _NCHUNK // 2, body, 0)
    plsc.subcore_barrier()
    pltpu.sync_copy(acc.at[pl.ds(s * _RPS, _RPS)],
                    out_hbm.at[c, pl.ds(s * _RPS, _RPS)])


def _type_of(i):
    return jnp.where(i < 4, 0, jnp.where(i < 7, 1, 2))


def _tc1_body(x_ref, w_ref, b_ref, d0_ref, d1_ref, x1_ref, xs_ref, rdeg_ref):
    deg = d0_ref[0, 0] + d1_ref[0, 0] + 1.0
    rdeg = lax.rsqrt(deg)
    rdeg_ref[0, 0] = rdeg
    x1 = jnp.dot(x_ref[...], w_ref[0], preferred_element_type=jnp.float32)
    x1 = x1 + b_ref[0]
    x1_ref[...] = x1
    xs_ref[...] = x1 * rdeg[:, None]


def _tc2_body(x1_ref, p0_ref, p1_ref, rdeg_ref, w_ref, b_ref, x2_ref, xs_ref):
    rdeg = rdeg_ref[0, 0][:, None]
    h = jnp.maximum(x1_ref[...] + rdeg * (p0_ref[...] + p1_ref[...]), 0.0)
    x2 = jnp.dot(h, w_ref[0], preferred_element_type=jnp.float32) + b_ref[0]
    x2_ref[...] = x2
    xs_ref[...] = x2 * rdeg


def _tc3_body(x2_ref, p0_ref, p1_ref, rdeg_ref, out_ref):
    rdeg = rdeg_ref[0, 0][:, None]
    out_ref[...] = x2_ref[...] + rdeg * (p0_ref[...] + p1_ref[...])


_row_spec = pl.BlockSpec((_BLK, _D), lambda i: (i, 0))
_w_spec = pl.BlockSpec((1, _D, _D), lambda i: (_type_of(i), 0, 0))
_b_spec = pl.BlockSpec((1, 1, _D), lambda i: (_type_of(i), 0, 0))
_vec_spec = pl.BlockSpec((1, 1, _BLK), lambda i: (i, 0, 0))

_tc1 = pl.pallas_call(
    _tc1_body,
    grid=(_GRID,),
    in_specs=[_row_spec, _w_spec, _b_spec, _vec_spec, _vec_spec],
    out_specs=[_row_spec, _row_spec, _vec_spec],
    out_shape=[
        jax.ShapeDtypeStruct((_N, _D), jnp.float32),
        jax.ShapeDtypeStruct((_N, _D), jnp.float32),
        jax.ShapeDtypeStruct((_GRID, 1, _BLK), jnp.float32),
    ],
)

_tc2 = pl.pallas_call(
    _tc2_body,
    grid=(_GRID,),
    in_specs=[_row_spec, _row_spec, _row_spec, _vec_spec, _w_spec, _b_spec],
    out_specs=[_row_spec, _row_spec],
    out_shape=[
        jax.ShapeDtypeStruct((_N, _D), jnp.float32),
        jax.ShapeDtypeStruct((_N, _D), jnp.float32),
    ],
)

_tc3 = pl.pallas_call(
    _tc3_body,
    grid=(_GRID,),
    in_specs=[_row_spec, _row_spec, _row_spec, _vec_spec],
    out_specs=_row_spec,
    out_shape=jax.ShapeDtypeStruct((_N, _D), jnp.float32),
)


def kernel(x0, x1, x2, edge_index0, edge_index1, edge_index2,
           W0_0, b0_0, W0_1, b0_1, W0_2, b0_2,
           W1_0, b1_0, W1_1, b1_1, W1_2, b1_2):
    x_cat = jnp.concatenate([x0, x1, x2], axis=0)
    W0 = jnp.stack([W0_0, W0_1, W0_2])
    b0 = jnp.stack([b0_0, b0_1, b0_2])[:, None, :]
    W1 = jnp.stack([W1_0, W1_1, W1_2])
    b1 = jnp.stack([b1_0, b1_1, b1_2])[:, None, :]

    src = jnp.concatenate([edge_index0[0], edge_index1[0], edge_index2[0]])
    dst = jnp.concatenate([edge_index0[1], edge_index1[1], edge_index2[1]])
    npad = _E_PAD - _E_TOT
    src_p = jnp.concatenate([src, jnp.zeros((npad,), jnp.int32)])
    dst_p = jnp.concatenate([dst, jnp.full((npad,), _N_PAD - 1, jnp.int32)])
    dst_w = dst_p.reshape(_NW, _NCHUNK, _CHUNK)
    packed = (src_p | (dst_p << 16)).reshape(_NW, _NCHUNK, _CHUNK)

    zeros1 = jnp.zeros((_N_PAD_DEG,), jnp.float32)
    zeros2 = jnp.zeros((_N_PAD, _D), jnp.float32)

    degp = _sc_degree(dst_w, zeros1)
    d0 = degp[0, :_N].reshape(_GRID, 1, _BLK)
    d1 = degp[1, :_N].reshape(_GRID, 1, _BLK)

    x1_full, xs1, rdeg_r = _tc1(x_cat, W0, b0, d0, d1)

    p = _sc_message(packed, xs1, zeros2)
    x2_full, xs2 = _tc2(x1_full, p[0, :_N], p[1, :_N], rdeg_r, W1, b1)

    q = _sc_message(packed, xs2, zeros2)
    return _tc3(x2_full, q[0, :_N], q[1, :_N], rdeg_r)
